# baseline (speedup 1.0000x reference)
import functools

import jax
import jax.numpy as jnp
from jax import lax
from jax.experimental import pallas as pl
from jax.experimental.pallas import tpu as pltpu
from jax.experimental.pallas import tpu_sc as plsc

N = 10000
NPAD = 10240
E = 320000
DIN = 128
DH = 256
DOUT = 64
B = 256
NB = NPAD // B
NC = 2
NS = 16
CHUNK = 128
RPS = NPAD // NS
W = 128
CH1 = 80
CH2 = 160
EPAD = NC * NS * CH1 * CHUNK
IBLK = 16


def _sc_segsum(table, src4, dst4, ch):
  mesh = plsc.VectorSubcoreMesh(core_axis_name="c", subcore_axis_name="s")

  @functools.partial(
      pl.kernel,
      out_type=jax.ShapeDtypeStruct((2 * NPAD, W), jnp.float32),
      mesh=mesh,
      scratch_types=[
          pltpu.VMEM((IBLK, CHUNK), jnp.int32),
          pltpu.VMEM((IBLK, CHUNK), jnp.int32),
          pltpu.VMEM((CHUNK, W), jnp.float32),
          pltpu.VMEM((CHUNK, W), jnp.float32),
          pltpu.VMEM_SHARED((NPAD, W), jnp.float32),
          pltpu.SemaphoreType.DMA,
      ],
  )
  def k(table_hbm, src_hbm, dst_hbm, out_hbm, src_v, dst_v, rows0, rows1,
        acc, sem_g):
    c = lax.axis_index("c")
    s = lax.axis_index("s")

    def zrow(r, carry):
      for kk in range(W // 16):
        rows0[r, pl.ds(kk * 16, 16)] = jnp.zeros((16,), jnp.float32)
      return carry
    lax.fori_loop(0, CHUNK, zrow, 0)
    for q in range(RPS // CHUNK):
      pltpu.sync_copy(rows0, acc.at[pl.ds(s * RPS + q * CHUNK, CHUNK)])
    plsc.subcore_barrier()

    def start_g(j, buf):
      pltpu.async_copy(table_hbm.at[src_v.at[j]], buf, sem_g)

    def wait_g(buf):
      pltpu.make_async_copy(table_hbm.at[src_v.at[0]], buf, sem_g).wait()

    def scat(j, buf):
      pltpu.sync_copy(buf, acc.at[dst_v.at[j]], add=True)

    for t in range(ch // IBLK):
      pltpu.sync_copy(src_hbm.at[c, s, pl.ds(t * IBLK, IBLK)], src_v)
      pltpu.sync_copy(dst_hbm.at[c, s, pl.ds(t * IBLK, IBLK)], dst_v)
      start_g(0, rows0)

      def pair(i, carry):
        j0 = 2 * i
        wait_g(rows0)
        start_g(j0 + 1, rows1)
        scat(j0, rows0)
        wait_g(rows1)
        start_g(j0 + 2, rows0)
        scat(j0 + 1, rows1)
        return carry
      lax.fori_loop(0, IBLK // 2 - 1, pair, 0)
      wait_g(rows0)
      start_g(IBLK - 1, rows1)
      scat(IBLK - 2, rows0)
      wait_g(rows1)
      scat(IBLK - 1, rows1)
    plsc.subcore_barrier()

    for q in range(RPS // CHUNK):
      pltpu.sync_copy(acc.at[pl.ds(s * RPS + q * CHUNK, CHUNK)], rows0)
      pltpu.sync_copy(rows0,
                      out_hbm.at[pl.ds(c * NPAD + s * RPS + q * CHUNK, CHUNK)])

  return k(table, src4, dst4)


_TC_PARAMS = pltpu.CompilerParams(dimension_semantics=("arbitrary",))
_FULL = lambda shape: pl.BlockSpec(shape, lambda i: (0, 0))


def _mlp1(Xp, agg, Wa, ba, Wb, bb):
  def body(x_ref, aa_ref, ab_ref, wa_ref, ba_ref, wb_ref, bb_ref,
           h_ref, sums_ref):
    i = pl.program_id(0)
    h0 = x_ref[...] + aa_ref[...] + ab_ref[...]
    h = jnp.maximum(jnp.dot(h0, wa_ref[...],
                            preferred_element_type=jnp.float32) + ba_ref[...], 0.0)
    h = jnp.maximum(jnp.dot(h, wb_ref[...],
                            preferred_element_type=jnp.float32) + bb_ref[...], 0.0)
    h_ref[...] = h
    rows = i * B + lax.broadcasted_iota(jnp.int32, (B, 1), 0)
    hm = jnp.where(rows < N, h, 0.0)
    upd = jnp.concatenate([jnp.sum(hm, 0)[None], jnp.sum(hm * hm, 0)[None],
                           jnp.zeros((6, DH), jnp.float32)], axis=0)
    @pl.when(i == 0)
    def _():
      sums_ref[...] = jnp.zeros_like(sums_ref)
    sums_ref[...] += upd

  return pl.pallas_call(
      body,
      grid=(NB,),
      in_specs=[
          pl.BlockSpec((B, DIN), lambda i: (i, 0)),
          pl.BlockSpec((B, DIN), lambda i: (i, 0)),
          pl.BlockSpec((B, DIN), lambda i: (NB + i, 0)),
          _FULL((DIN, DH)), _FULL((1, DH)), _FULL((DH, DH)), _FULL((1, DH)),
      ],
      out_specs=[
          pl.BlockSpec((B, DH), lambda i: (i, 0)),
          pl.BlockSpec((8, DH), lambda i: (0, 0)),
      ],
      out_shape=[
          jax.ShapeDtypeStruct((NPAD, DH), jnp.float32),
          jax.ShapeDtypeStruct((8, DH), jnp.float32),
      ],
      compiler_params=_TC_PARAMS,
  )(Xp, agg, agg, Wa, ba, Wb, bb)


def _mlp2(H1r, agg, Wa, ba, Wb, bb):
  def body(xl_ref, xr_ref, al_ref, ar_ref, wa_ref, ba_ref, wb_ref, bb_ref,
           h_ref, sums_ref):
    i = pl.program_id(0)
    h0 = jnp.concatenate([xl_ref[...] + al_ref[...],
                          xr_ref[...] + ar_ref[...]], axis=1)
    h = jnp.maximum(jnp.dot(h0, wa_ref[...],
                            preferred_element_type=jnp.float32) + ba_ref[...], 0.0)
    h = jnp.maximum(jnp.dot(h, wb_ref[...],
                            preferred_element_type=jnp.float32) + bb_ref[...], 0.0)
    h_ref[...] = h
    rows = i * B + lax.broadcasted_iota(jnp.int32, (B, 1), 0)
    hm = jnp.where(rows < N, h, 0.0)
    upd = jnp.concatenate([jnp.sum(hm, 0)[None], jnp.sum(hm * hm, 0)[None],
                           jnp.zeros((6, DH), jnp.float32)], axis=0)
    @pl.when(i == 0)
    def _():
      sums_ref[...] = jnp.zeros_like(sums_ref)
    sums_ref[...] += upd

  halfspec_lo = pl.BlockSpec((B, DH // 2), lambda i: (i, 0))
  halfspec_hi = pl.BlockSpec((B, DH // 2), lambda i: (NB + i, 0))
  return pl.pallas_call(
      body,
      grid=(NB,),
      in_specs=[halfspec_lo, halfspec_hi, halfspec_lo, halfspec_hi,
                _FULL((DH, DH)), _FULL((1, DH)), _FULL((DH, DH)), _FULL((1, DH))],
      out_specs=[
          pl.BlockSpec((B, DH), lambda i: (i, 0)),
          pl.BlockSpec((8, DH), lambda i: (0, 0)),
      ],
      out_shape=[
          jax.ShapeDtypeStruct((NPAD, DH), jnp.float32),
          jax.ShapeDtypeStruct((8, DH), jnp.float32),
      ],
      compiler_params=_TC_PARAMS,
  )(H1r, H1r, agg, agg, Wa, ba, Wb, bb)


def _bn_split(h_pre, sums, g, be):
  def body(h_ref, s_ref, g_ref, be_ref, o_ref):
    mean = s_ref[0:1, :] / N
    var = s_ref[1:2, :] / N - mean * mean
    scale = g_ref[...] * lax.rsqrt(var + 1e-5)
    shift = be_ref[...] - mean * scale
    o_ref[...] = h_ref[...] * scale + shift

  return pl.pallas_call(
      body,
      grid=(2, NB),
      in_specs=[pl.BlockSpec((B, DH // 2), lambda p, i: (i, p)),
                pl.BlockSpec((8, DH // 2), lambda p, i: (0, p)),
                pl.BlockSpec((1, DH // 2), lambda p, i: (0, p)),
                pl.BlockSpec((1, DH // 2), lambda p, i: (0, p))],
      out_specs=pl.BlockSpec((B, DH // 2), lambda p, i: (p * NB + i, 0)),
      out_shape=jax.ShapeDtypeStruct((2 * NPAD, DH // 2), jnp.float32),
      compiler_params=pltpu.CompilerParams(
          dimension_semantics=("arbitrary", "arbitrary")),
  )(h_pre, sums, g, be)


def _head(h_pre, sums, g, be, maskp, W3, b3):
  def body(h_ref, s_ref, g_ref, be_ref, m_ref, w3_ref, b3_ref, o_ref):
    mean = s_ref[0:1, :] / N
    var = s_ref[1:2, :] / N - mean * mean
    scale = g_ref[...] * lax.rsqrt(var + 1e-5)
    shift = be_ref[...] - mean * scale
    hv = h_ref[...] * scale + shift
    hd = hv * m_ref[...]
    z = jnp.dot(hd, w3_ref[...], preferred_element_type=jnp.float32) + b3_ref[...]
    zmax = jnp.max(z, axis=1, keepdims=True)
    lse = jnp.log(jnp.sum(jnp.exp(z - zmax), axis=1, keepdims=True)) + zmax
    o_ref[...] = z - lse

  return pl.pallas_call(
      body,
      grid=(NB,),
      in_specs=[pl.BlockSpec((B, DH), lambda i: (i, 0)),
                _FULL((8, DH)), _FULL((1, DH)), _FULL((1, DH)),
                pl.BlockSpec((B, DH), lambda i: (i, 0)),
                _FULL((DH, DOUT)), _FULL((1, DOUT))],
      out_specs=pl.BlockSpec((B, DOUT), lambda i: (i, 0)),
      out_shape=jax.ShapeDtypeStruct((NPAD, DOUT), jnp.float32),
      compiler_params=_TC_PARAMS,
  )(h_pre, sums, g, be, maskp, W3, b3)


def kernel(X, edge_index, W1a, b1a, W1b, b1b, g1, be1,
           W2a, b2a, W2b, b2b, g2, be2, W3, b3):
  src = edge_index[0].astype(jnp.int32)
  dst = edge_index[1].astype(jnp.int32)
  pad_ids = lax.iota(jnp.int32, EPAD - E)
  srcp = jnp.concatenate([src, pad_ids % N])
  dstp = jnp.concatenate([dst, N + pad_ids % (NPAD - N)])
  src1_4 = srcp.reshape(NC, NS, CH1, CHUNK)
  dst1_4 = dstp.reshape(NC, NS, CH1, CHUNK)

  src2_4 = jnp.stack([srcp, srcp + NPAD]).reshape(NC, NS, CH2, CHUNK)
  dst2_4 = jnp.stack([dstp, dstp]).reshape(NC, NS, CH2, CHUNK)

  Xp = jnp.pad(X, ((0, NPAD - N), (0, 0)))

  b1a2, b1b2 = b1a[None, :], b1b[None, :]
  b2a2, b2b2 = b2a[None, :], b2b[None, :]
  g1r, be1r = g1[None, :], be1[None, :]
  g2r, be2r = g2[None, :], be2[None, :]
  b3r = b3[None, :]

  agg1 = _sc_segsum(Xp, src1_4, dst1_4, CH1)
  h1_pre, sums1 = _mlp1(Xp, agg1, W1a, b1a2, W1b, b1b2)
  H1r = _bn_split(h1_pre, sums1, g1r, be1r)

  agg2 = _sc_segsum(H1r, src2_4, dst2_4, CH2)
  h2_pre, sums2 = _mlp2(H1r, agg2, W2a, b2a2, W2b, b2b2)

  mask = jax.random.bernoulli(jax.random.key(123), 0.5, (N, DH))
  maskp = jnp.pad(mask.astype(jnp.float32) * 2.0, ((0, NPAD - N), (0, 0)))
  out = _head(h2_pre, sums2, g2r, be2r, maskp, W3, b3r)
  return out[:N]


# speedup vs baseline: 1.0408x; 1.0408x over previous
import functools

import jax
import jax.numpy as jnp
from jax import lax
from jax.experimental import pallas as pl
from jax.experimental.pallas import tpu as pltpu
from jax.experimental.pallas import tpu_sc as plsc

N = 10000
NPAD = 10240
E = 320000
DIN = 128
DH = 256
DOUT = 64
B = 256
NB = NPAD // B
NC = 2
NS = 16
CHUNK = 128
RPS = NPAD // NS
W = 128
CH1 = 80
CH2 = 160
EPAD = NC * NS * CH1 * CHUNK
IBLK = 16


def _sc_segsum(table, src4, dst4, ch):
  mesh = plsc.VectorSubcoreMesh(core_axis_name="c", subcore_axis_name="s")

  @functools.partial(
      pl.kernel,
      out_type=jax.ShapeDtypeStruct((2 * NPAD, W), jnp.float32),
      mesh=mesh,
      scratch_types=[
          pltpu.VMEM((IBLK, CHUNK), jnp.int32),
          pltpu.VMEM((IBLK, CHUNK), jnp.int32),
          pltpu.VMEM((CHUNK, W), jnp.float32),
          pltpu.VMEM((CHUNK, W), jnp.float32),
          pltpu.VMEM_SHARED((NPAD, W), jnp.float32),
          pltpu.SemaphoreType.DMA,
      ],
  )
  def k(table_hbm, src_hbm, dst_hbm, out_hbm, src_v, dst_v, rows0, rows1,
        acc, sem_g):
    c = lax.axis_index("c")
    s = lax.axis_index("s")

    def zrow(r, carry):
      for kk in range(W // 16):
        rows0[r, pl.ds(kk * 16, 16)] = jnp.zeros((16,), jnp.float32)
      return carry
    lax.fori_loop(0, CHUNK, zrow, 0)
    for q in range(RPS // CHUNK):
      pltpu.sync_copy(rows0, acc.at[pl.ds(s * RPS + q * CHUNK, CHUNK)])
    plsc.subcore_barrier()

    def start_g(j, buf):
      pltpu.async_copy(table_hbm.at[src_v.at[j]], buf, sem_g)

    def wait_g(buf):
      pltpu.make_async_copy(table_hbm.at[src_v.at[0]], buf, sem_g).wait()

    def scat(j, buf):
      pltpu.sync_copy(buf, acc.at[dst_v.at[j]], add=True)

    for t in range(ch // IBLK):
      pltpu.sync_copy(src_hbm.at[c, s, pl.ds(t * IBLK, IBLK)], src_v)
      pltpu.sync_copy(dst_hbm.at[c, s, pl.ds(t * IBLK, IBLK)], dst_v)
      start_g(0, rows0)

      def pair(i, carry):
        j0 = 2 * i
        wait_g(rows0)
        start_g(j0 + 1, rows1)
        scat(j0, rows0)
        wait_g(rows1)
        start_g(j0 + 2, rows0)
        scat(j0 + 1, rows1)
        return carry
      lax.fori_loop(0, IBLK // 2 - 1, pair, 0)
      wait_g(rows0)
      start_g(IBLK - 1, rows1)
      scat(IBLK - 2, rows0)
      wait_g(rows1)
      scat(IBLK - 1, rows1)
    plsc.subcore_barrier()

    for q in range(RPS // CHUNK):
      pltpu.sync_copy(acc.at[pl.ds(s * RPS + q * CHUNK, CHUNK)], rows0)
      pltpu.sync_copy(rows0,
                      out_hbm.at[pl.ds(c * NPAD + s * RPS + q * CHUNK, CHUNK)])

  return k(table, src4, dst4)


_TC_PARAMS = pltpu.CompilerParams(dimension_semantics=("arbitrary",))
_FULL = lambda shape: pl.BlockSpec(shape, lambda i: (0, 0))


def _mlp_block(h0, wa_ref, ba_ref, wb_ref, bb_ref, i, sums):
  h = jnp.maximum(jnp.dot(h0, wa_ref[...],
                          preferred_element_type=jnp.float32) + ba_ref[...], 0.0)
  h = jnp.maximum(jnp.dot(h, wb_ref[...],
                          preferred_element_type=jnp.float32) + bb_ref[...], 0.0)
  rows = i * B + lax.broadcasted_iota(jnp.int32, (B, 1), 0)
  hm = jnp.where(rows < N, h, 0.0)
  @pl.when(i == 0)
  def _():
    sums[...] = jnp.zeros_like(sums)
  sums[...] += jnp.concatenate(
      [jnp.sum(hm, 0)[None], jnp.sum(hm * hm, 0)[None],
       jnp.zeros((6, DH), jnp.float32)], axis=0)
  return h


def _bn_coeffs(sums, g_ref, be_ref):
  mean = sums[0:1, :] / N
  var = sums[1:2, :] / N - mean * mean
  scale = g_ref[...] * lax.rsqrt(var + 1e-5)
  shift = be_ref[...] - mean * scale
  return scale, shift


def _layer1(Xp, agg, Wa, ba, Wb, bb, g, be):
  def body(x_ref, aa_ref, ab_ref, wa_ref, ba_ref, wb_ref, bb_ref,
           g_ref, be_ref, o_ref, hbuf, sums):
    i = pl.program_id(0)

    @pl.when(i < NB)
    def _():
      h0 = x_ref[...] + aa_ref[...] + ab_ref[...]
      hbuf[pl.ds(i * B, B), :] = _mlp_block(h0, wa_ref, ba_ref, wb_ref,
                                            bb_ref, i, sums)

    @pl.when(i >= NB)
    def _():
      k = i - NB
      p = k // NB
      r = k % NB
      scale, shift = _bn_coeffs(sums, g_ref, be_ref)
      hv = hbuf[pl.ds(r * B, B), :] * scale + shift
      o_ref[...] = jnp.where(p == 0, hv[:, :DH // 2], hv[:, DH // 2:])

  clamp = lambda i: (jnp.minimum(i, NB - 1), 0)
  clamp_hi = lambda i: (NB + jnp.minimum(i, NB - 1), 0)
  return pl.pallas_call(
      body,
      grid=(3 * NB,),
      in_specs=[
          pl.BlockSpec((B, DIN), clamp),
          pl.BlockSpec((B, DIN), clamp),
          pl.BlockSpec((B, DIN), clamp_hi),
          _FULL((DIN, DH)), _FULL((1, DH)), _FULL((DH, DH)), _FULL((1, DH)),
          _FULL((1, DH)), _FULL((1, DH)),
      ],
      out_specs=pl.BlockSpec(
          (B, DH // 2),
          lambda i: (jnp.where(i < NB, 0, i - NB), 0)),
      out_shape=jax.ShapeDtypeStruct((2 * NPAD, DH // 2), jnp.float32),
      scratch_shapes=[pltpu.VMEM((NPAD, DH), jnp.float32),
                      pltpu.VMEM((8, DH), jnp.float32)],
      compiler_params=_TC_PARAMS,
  )(Xp, agg, agg, Wa, ba, Wb, bb, g, be)


def _layer2(H1r, agg, Wa, ba, Wb, bb, g, be, maskp, W3, b3):
  def body(xl_ref, xr_ref, al_ref, ar_ref, wa_ref, ba_ref, wb_ref, bb_ref,
           g_ref, be_ref, m_ref, w3_ref, b3_ref, o_ref, hbuf, sums):
    i = pl.program_id(0)

    @pl.when(i < NB)
    def _():
      h0 = jnp.concatenate([xl_ref[...] + al_ref[...],
                            xr_ref[...] + ar_ref[...]], axis=1)
      hbuf[pl.ds(i * B, B), :] = _mlp_block(h0, wa_ref, ba_ref, wb_ref,
                                            bb_ref, i, sums)

    @pl.when(i >= NB)
    def _():
      r = i - NB
      scale, shift = _bn_coeffs(sums, g_ref, be_ref)
      hv = hbuf[pl.ds(r * B, B), :] * scale + shift
      hd = hv * m_ref[...]
      z = jnp.dot(hd, w3_ref[...],
                  preferred_element_type=jnp.float32) + b3_ref[...]
      zmax = jnp.max(z, axis=1, keepdims=True)
      lse = jnp.log(jnp.sum(jnp.exp(z - zmax), axis=1, keepdims=True)) + zmax
      o_ref[...] = z - lse

  clamp = lambda i: (jnp.minimum(i, NB - 1), 0)
  clamp_hi = lambda i: (NB + jnp.minimum(i, NB - 1), 0)
  phase1 = lambda i: (jnp.where(i < NB, 0, i - NB), 0)
  return pl.pallas_call(
      body,
      grid=(2 * NB,),
      in_specs=[
          pl.BlockSpec((B, DH // 2), clamp),
          pl.BlockSpec((B, DH // 2), clamp_hi),
          pl.BlockSpec((B, DH // 2), clamp),
          pl.BlockSpec((B, DH // 2), clamp_hi),
          _FULL((DH, DH)), _FULL((1, DH)), _FULL((DH, DH)), _FULL((1, DH)),
          _FULL((1, DH)), _FULL((1, DH)),
          pl.BlockSpec((B, DH), phase1),
          _FULL((DH, DOUT)), _FULL((1, DOUT)),
      ],
      out_specs=pl.BlockSpec((B, DOUT), phase1),
      out_shape=jax.ShapeDtypeStruct((NPAD, DOUT), jnp.float32),
      scratch_shapes=[pltpu.VMEM((NPAD, DH), jnp.float32),
                      pltpu.VMEM((8, DH), jnp.float32)],
      compiler_params=_TC_PARAMS,
  )(H1r, H1r, agg, agg, Wa, ba, Wb, bb, g, be, maskp, W3, b3)


def kernel(X, edge_index, W1a, b1a, W1b, b1b, g1, be1,
           W2a, b2a, W2b, b2b, g2, be2, W3, b3):
  src = edge_index[0].astype(jnp.int32)
  dst = edge_index[1].astype(jnp.int32)
  pad_ids = lax.iota(jnp.int32, EPAD - E)
  srcp = jnp.concatenate([src, pad_ids % N])
  dstp = jnp.concatenate([dst, N + pad_ids % (NPAD - N)])
  src1_4 = srcp.reshape(NC, NS, CH1, CHUNK)
  dst1_4 = dstp.reshape(NC, NS, CH1, CHUNK)

  src2_4 = jnp.stack([srcp, srcp + NPAD]).reshape(NC, NS, CH2, CHUNK)
  dst2_4 = jnp.stack([dstp, dstp]).reshape(NC, NS, CH2, CHUNK)

  Xp = jnp.pad(X, ((0, NPAD - N), (0, 0)))

  b1a2, b1b2 = b1a[None, :], b1b[None, :]
  b2a2, b2b2 = b2a[None, :], b2b[None, :]
  g1r, be1r = g1[None, :], be1[None, :]
  g2r, be2r = g2[None, :], be2[None, :]
  b3r = b3[None, :]

  agg1 = _sc_segsum(Xp, src1_4, dst1_4, CH1)
  H1r = _layer1(Xp, agg1, W1a, b1a2, W1b, b1b2, g1r, be1r)

  agg2 = _sc_segsum(H1r, src2_4, dst2_4, CH2)
  mask = jax.random.bernoulli(jax.random.key(123), 0.5, (N, DH))
  maskp = jnp.pad(mask.astype(jnp.float32) * 2.0, ((0, NPAD - N), (0, 0)))
  out = _layer2(H1r, agg2, W2a, b2a2, W2b, b2b2, g2r, be2r, maskp, W3, b3r)
  return out[:N]


# speedup vs baseline: 1.1388x; 1.0942x over previous
import functools

import jax
import jax.numpy as jnp
from jax import lax
from jax.experimental import pallas as pl
from jax.experimental.pallas import tpu as pltpu
from jax.experimental.pallas import tpu_sc as plsc

N = 10000
NPAD = 10240
E = 320000
DIN = 128
DH = 256
DOUT = 64
B = 512
NB = NPAD // B
NC = 2
NS = 16
CHUNK = 128
RPS = NPAD // NS
W = 128
CH1 = 80
CH2 = 160
EPAD = NC * NS * CH1 * CHUNK
IBLK = 16


def _sc_segsum(table, src4, dst4, ch):
  mesh = plsc.VectorSubcoreMesh(core_axis_name="c", subcore_axis_name="s")

  @functools.partial(
      pl.kernel,
      out_type=jax.ShapeDtypeStruct((2 * NPAD, W), jnp.float32),
      mesh=mesh,
      scratch_types=[
          pltpu.VMEM((IBLK, CHUNK), jnp.int32),
          pltpu.VMEM((IBLK, CHUNK), jnp.int32),
          pltpu.VMEM((CHUNK, W), jnp.float32),
          pltpu.VMEM((CHUNK, W), jnp.float32),
          pltpu.VMEM_SHARED((NPAD, W), jnp.float32),
          pltpu.SemaphoreType.DMA,
      ],
  )
  def k(table_hbm, src_hbm, dst_hbm, out_hbm, src_v, dst_v, rows0, rows1,
        acc, sem_g):
    c = lax.axis_index("c")
    s = lax.axis_index("s")

    def zrow(r, carry):
      for kk in range(W // 16):
        rows0[r, pl.ds(kk * 16, 16)] = jnp.zeros((16,), jnp.float32)
      return carry
    lax.fori_loop(0, CHUNK, zrow, 0)
    for q in range(RPS // CHUNK):
      pltpu.sync_copy(rows0, acc.at[pl.ds(s * RPS + q * CHUNK, CHUNK)])
    plsc.subcore_barrier()

    def start_g(j, buf):
      pltpu.async_copy(table_hbm.at[src_v.at[j]], buf, sem_g)

    def wait_g(buf):
      pltpu.make_async_copy(table_hbm.at[src_v.at[0]], buf, sem_g).wait()

    def scat(j, buf):
      pltpu.sync_copy(buf, acc.at[dst_v.at[j]], add=True)

    for t in range(ch // IBLK):
      pltpu.sync_copy(src_hbm.at[c, s, pl.ds(t * IBLK, IBLK)], src_v)
      pltpu.sync_copy(dst_hbm.at[c, s, pl.ds(t * IBLK, IBLK)], dst_v)
      start_g(0, rows0)

      def pair(i, carry):
        j0 = 2 * i
        wait_g(rows0)
        start_g(j0 + 1, rows1)
        scat(j0, rows0)
        wait_g(rows1)
        start_g(j0 + 2, rows0)
        scat(j0 + 1, rows1)
        return carry
      lax.fori_loop(0, IBLK // 2 - 1, pair, 0)
      wait_g(rows0)
      start_g(IBLK - 1, rows1)
      scat(IBLK - 2, rows0)
      wait_g(rows1)
      scat(IBLK - 1, rows1)
    plsc.subcore_barrier()

    for q in range(RPS // CHUNK):
      pltpu.sync_copy(acc.at[pl.ds(s * RPS + q * CHUNK, CHUNK)], rows0)
      pltpu.sync_copy(rows0,
                      out_hbm.at[pl.ds(c * NPAD + s * RPS + q * CHUNK, CHUNK)])

  return k(table, src4, dst4)


_TC_PARAMS = pltpu.CompilerParams(dimension_semantics=("arbitrary",))
_FULL = lambda shape: pl.BlockSpec(shape, lambda i: (0, 0))


def _mlp_block(h0, wa_ref, ba_ref, wb_ref, bb_ref, i, sums):
  h = jnp.maximum(jnp.dot(h0.astype(jnp.bfloat16), wa_ref[...],
                          preferred_element_type=jnp.float32) + ba_ref[...], 0.0)
  h = jnp.maximum(jnp.dot(h.astype(jnp.bfloat16), wb_ref[...],
                          preferred_element_type=jnp.float32) + bb_ref[...], 0.0)
  rows = i * B + lax.broadcasted_iota(jnp.int32, (B, 1), 0)
  hm = jnp.where(rows < N, h, 0.0)
  @pl.when(i == 0)
  def _():
    sums[...] = jnp.zeros_like(sums)
  sums[...] += jnp.concatenate(
      [jnp.sum(hm, 0)[None], jnp.sum(hm * hm, 0)[None],
       jnp.zeros((6, DH), jnp.float32)], axis=0)
  return h


def _bn_coeffs(sums, g_ref, be_ref):
  mean = sums[0:1, :] / N
  var = sums[1:2, :] / N - mean * mean
  scale = g_ref[...] * lax.rsqrt(var + 1e-5)
  shift = be_ref[...] - mean * scale
  return scale, shift


def _layer1(Xp, agg, Wa, ba, Wb, bb, g, be):
  def body(x_ref, aa_ref, ab_ref, wa_ref, ba_ref, wb_ref, bb_ref,
           g_ref, be_ref, o_ref, hbuf, sums):
    i = pl.program_id(0)

    @pl.when(i < NB)
    def _():
      h0 = x_ref[...] + aa_ref[...] + ab_ref[...]
      hbuf[pl.ds(i * B, B), :] = _mlp_block(h0, wa_ref, ba_ref, wb_ref,
                                            bb_ref, i, sums)

    @pl.when(i >= NB)
    def _():
      k = i - NB
      p = k // NB
      r = k % NB
      scale, shift = _bn_coeffs(sums, g_ref, be_ref)
      hv = hbuf[pl.ds(r * B, B), :] * scale + shift
      o_ref[...] = jnp.where(p == 0, hv[:, :DH // 2], hv[:, DH // 2:])

  clamp = lambda i: (jnp.minimum(i, NB - 1), 0)
  clamp_hi = lambda i: (NB + jnp.minimum(i, NB - 1), 0)
  return pl.pallas_call(
      body,
      grid=(3 * NB,),
      in_specs=[
          pl.BlockSpec((B, DIN), clamp),
          pl.BlockSpec((B, DIN), clamp),
          pl.BlockSpec((B, DIN), clamp_hi),
          _FULL((DIN, DH)), _FULL((1, DH)), _FULL((DH, DH)), _FULL((1, DH)),
          _FULL((1, DH)), _FULL((1, DH)),
      ],
      out_specs=pl.BlockSpec(
          (B, DH // 2),
          lambda i: (jnp.where(i < NB, 0, i - NB), 0)),
      out_shape=jax.ShapeDtypeStruct((2 * NPAD, DH // 2), jnp.float32),
      scratch_shapes=[pltpu.VMEM((NPAD, DH), jnp.float32),
                      pltpu.VMEM((8, DH), jnp.float32)],
      compiler_params=_TC_PARAMS,
  )(Xp, agg, agg, Wa, ba, Wb, bb, g, be)


def _layer2(H1r, agg, Wa, ba, Wb, bb, g, be, maskp, W3, b3):
  def body(xl_ref, xr_ref, al_ref, ar_ref, wa_ref, ba_ref, wb_ref, bb_ref,
           g_ref, be_ref, m_ref, w3_ref, b3_ref, o_ref, hbuf, sums):
    i = pl.program_id(0)

    @pl.when(i < NB)
    def _():
      h0 = jnp.concatenate([xl_ref[...] + al_ref[...],
                            xr_ref[...] + ar_ref[...]], axis=1)
      hbuf[pl.ds(i * B, B), :] = _mlp_block(h0, wa_ref, ba_ref, wb_ref,
                                            bb_ref, i, sums)

    @pl.when(i >= NB)
    def _():
      r = i - NB
      scale, shift = _bn_coeffs(sums, g_ref, be_ref)
      hv = hbuf[pl.ds(r * B, B), :] * scale + shift
      hd = jnp.where(m_ref[...], hv + hv, 0.0)
      z = jnp.dot(hd.astype(jnp.bfloat16), w3_ref[...],
                  preferred_element_type=jnp.float32) + b3_ref[...]
      zmax = jnp.max(z, axis=1, keepdims=True)
      lse = jnp.log(jnp.sum(jnp.exp(z - zmax), axis=1, keepdims=True)) + zmax
      o_ref[...] = z - lse

  clamp = lambda i: (jnp.minimum(i, NB - 1), 0)
  clamp_hi = lambda i: (NB + jnp.minimum(i, NB - 1), 0)
  phase1 = lambda i: (jnp.where(i < NB, 0, i - NB), 0)
  return pl.pallas_call(
      body,
      grid=(2 * NB,),
      in_specs=[
          pl.BlockSpec((B, DH // 2), clamp),
          pl.BlockSpec((B, DH // 2), clamp_hi),
          pl.BlockSpec((B, DH // 2), clamp),
          pl.BlockSpec((B, DH // 2), clamp_hi),
          _FULL((DH, DH)), _FULL((1, DH)), _FULL((DH, DH)), _FULL((1, DH)),
          _FULL((1, DH)), _FULL((1, DH)),
          pl.BlockSpec((B, DH), phase1),
          _FULL((DH, DOUT)), _FULL((1, DOUT)),
      ],
      out_specs=pl.BlockSpec((B, DOUT), phase1),
      out_shape=jax.ShapeDtypeStruct((NPAD, DOUT), jnp.float32),
      scratch_shapes=[pltpu.VMEM((NPAD, DH), jnp.float32),
                      pltpu.VMEM((8, DH), jnp.float32)],
      compiler_params=_TC_PARAMS,
  )(H1r, H1r, agg, agg, Wa, ba, Wb, bb, g, be, maskp, W3, b3)


def kernel(X, edge_index, W1a, b1a, W1b, b1b, g1, be1,
           W2a, b2a, W2b, b2b, g2, be2, W3, b3):
  src = edge_index[0].astype(jnp.int32)
  dst = edge_index[1].astype(jnp.int32)
  pad_ids = lax.iota(jnp.int32, EPAD - E)
  srcp = jnp.concatenate([src, pad_ids % N])
  dstp = jnp.concatenate([dst, N + pad_ids % (NPAD - N)])
  src1_4 = srcp.reshape(NC, NS, CH1, CHUNK)
  dst1_4 = dstp.reshape(NC, NS, CH1, CHUNK)

  src2_4 = jnp.stack([srcp, srcp + NPAD]).reshape(NC, NS, CH2, CHUNK)
  dst2_4 = jnp.stack([dstp, dstp]).reshape(NC, NS, CH2, CHUNK)

  Xp = jnp.pad(X, ((0, NPAD - N), (0, 0)))

  b1a2, b1b2 = b1a[None, :], b1b[None, :]
  b2a2, b2b2 = b2a[None, :], b2b[None, :]
  g1r, be1r = g1[None, :], be1[None, :]
  g2r, be2r = g2[None, :], be2[None, :]
  b3r = b3[None, :]

  bf = jnp.bfloat16
  agg1 = _sc_segsum(Xp, src1_4, dst1_4, CH1)
  H1r = _layer1(Xp, agg1, W1a.astype(bf), b1a2, W1b.astype(bf), b1b2,
                g1r, be1r)

  agg2 = _sc_segsum(H1r, src2_4, dst2_4, CH2)
  mask = jax.random.bernoulli(jax.random.key(123), 0.5, (N, DH))
  maskp = jnp.pad(mask, ((0, NPAD - N), (0, 0)))
  out = _layer2(H1r, agg2, W2a.astype(bf), b2a2, W2b.astype(bf), b2b2,
                g2r, be2r, maskp, W3.astype(bf), b3r)
  return out[:N]


# speedup vs baseline: 1.1508x; 1.0105x over previous
import functools

import jax
import jax.numpy as jnp
from jax import lax
from jax.experimental import pallas as pl
from jax.experimental.pallas import tpu as pltpu
from jax.experimental.pallas import tpu_sc as plsc

N = 10000
NPAD = 10240
E = 320000
DIN = 128
DH = 256
DOUT = 64
B = 512
NB = NPAD // B
NC = 2
NS = 16
CHUNK = 128
RPS = NPAD // NS
W = 128
CH1 = 80
CH2 = 160
EPAD = NC * NS * CH1 * CHUNK
IBLK = 16


def _sc_segsum(table, src4, dst4, ch):
  mesh = plsc.VectorSubcoreMesh(core_axis_name="c", subcore_axis_name="s")

  @functools.partial(
      pl.kernel,
      out_type=jax.ShapeDtypeStruct((2 * NPAD, W), jnp.float32),
      mesh=mesh,
      scratch_types=[
          pltpu.VMEM((IBLK, CHUNK), jnp.int32),
          pltpu.VMEM((IBLK, CHUNK), jnp.int32),
          pltpu.VMEM((CHUNK, W), jnp.float32),
          pltpu.VMEM((CHUNK, W), jnp.float32),
          pltpu.VMEM_SHARED((NPAD, W), jnp.float32),
          pltpu.SemaphoreType.DMA,
          pltpu.SemaphoreType.DMA,
      ],
  )
  def k(table_hbm, src_hbm, dst_hbm, out_hbm, src_v, dst_v, rows0, rows1,
        acc, sem_g, sem_z):
    c = lax.axis_index("c")
    s = lax.axis_index("s")

    def start_g(j, buf):
      pltpu.async_copy(table_hbm.at[src_v.at[j]], buf, sem_g)

    def wait_g(buf):
      pltpu.make_async_copy(table_hbm.at[src_v.at[0]], buf, sem_g).wait()

    def scat(j, buf):
      pltpu.sync_copy(buf, acc.at[dst_v.at[j]], add=True)

    def zrow(r, carry):
      for kk in range(W // 16):
        rows1[r, pl.ds(kk * 16, 16)] = jnp.zeros((16,), jnp.float32)
      return carry
    lax.fori_loop(0, CHUNK, zrow, 0)
    zdescs = [pltpu.async_copy(rows1,
                               acc.at[pl.ds(s * RPS + q * CHUNK, CHUNK)],
                               sem_z)
              for q in range(RPS // CHUNK)]
    pltpu.sync_copy(src_hbm.at[c, s, pl.ds(0, IBLK)], src_v)
    pltpu.sync_copy(dst_hbm.at[c, s, pl.ds(0, IBLK)], dst_v)
    start_g(0, rows0)
    for zd in zdescs:
      zd.wait()
    plsc.subcore_barrier()

    for t in range(ch // IBLK):
      if t > 0:
        pltpu.sync_copy(src_hbm.at[c, s, pl.ds(t * IBLK, IBLK)], src_v)
        pltpu.sync_copy(dst_hbm.at[c, s, pl.ds(t * IBLK, IBLK)], dst_v)
        start_g(0, rows0)

      def pair(i, carry):
        j0 = 2 * i
        wait_g(rows0)
        start_g(j0 + 1, rows1)
        scat(j0, rows0)
        wait_g(rows1)
        start_g(j0 + 2, rows0)
        scat(j0 + 1, rows1)
        return carry
      lax.fori_loop(0, IBLK // 2 - 1, pair, 0)
      wait_g(rows0)
      start_g(IBLK - 1, rows1)
      scat(IBLK - 2, rows0)
      wait_g(rows1)
      scat(IBLK - 1, rows1)
    plsc.subcore_barrier()

    pltpu.sync_copy(acc.at[pl.ds(s * RPS, RPS)],
                    out_hbm.at[pl.ds(c * NPAD + s * RPS, RPS)])

  return k(table, src4, dst4)


_TC_PARAMS = pltpu.CompilerParams(dimension_semantics=("arbitrary",))
_FULL = lambda shape: pl.BlockSpec(shape, lambda i: (0, 0))


def _mlp_block(h0, wa_ref, ba_ref, wb_ref, bb_ref, i, sums):
  h = jnp.maximum(jnp.dot(h0.astype(jnp.bfloat16), wa_ref[...],
                          preferred_element_type=jnp.float32) + ba_ref[...], 0.0)
  h = jnp.maximum(jnp.dot(h.astype(jnp.bfloat16), wb_ref[...],
                          preferred_element_type=jnp.float32) + bb_ref[...], 0.0)
  rows = i * B + lax.broadcasted_iota(jnp.int32, (B, 1), 0)
  hm = jnp.where(rows < N, h, 0.0)
  @pl.when(i == 0)
  def _():
    sums[...] = jnp.zeros_like(sums)
  sums[...] += jnp.concatenate(
      [jnp.sum(hm, 0)[None], jnp.sum(hm * hm, 0)[None],
       jnp.zeros((6, DH), jnp.float32)], axis=0)
  return h


def _bn_coeffs(sums, g_ref, be_ref):
  mean = sums[0:1, :] / N
  var = sums[1:2, :] / N - mean * mean
  scale = g_ref[...] * lax.rsqrt(var + 1e-5)
  shift = be_ref[...] - mean * scale
  return scale, shift


def _layer1(Xp, agg, Wa, ba, Wb, bb, g, be):
  def body(x_ref, aa_ref, ab_ref, wa_ref, ba_ref, wb_ref, bb_ref,
           g_ref, be_ref, o_ref, hbuf, sums):
    i = pl.program_id(0)

    @pl.when(i < NB)
    def _():
      h0 = x_ref[...] + aa_ref[...] + ab_ref[...]
      hbuf[pl.ds(i * B, B), :] = _mlp_block(h0, wa_ref, ba_ref, wb_ref,
                                            bb_ref, i, sums)

    @pl.when(i >= NB)
    def _():
      k = i - NB
      p = k // NB
      r = k % NB
      scale, shift = _bn_coeffs(sums, g_ref, be_ref)
      hv = hbuf[pl.ds(r * B, B), :] * scale + shift
      o_ref[...] = jnp.where(p == 0, hv[:, :DH // 2], hv[:, DH // 2:])

  clamp = lambda i: (jnp.minimum(i, NB - 1), 0)
  clamp_hi = lambda i: (NB + jnp.minimum(i, NB - 1), 0)
  return pl.pallas_call(
      body,
      grid=(3 * NB,),
      in_specs=[
          pl.BlockSpec((B, DIN), clamp),
          pl.BlockSpec((B, DIN), clamp),
          pl.BlockSpec((B, DIN), clamp_hi),
          _FULL((DIN, DH)), _FULL((1, DH)), _FULL((DH, DH)), _FULL((1, DH)),
          _FULL((1, DH)), _FULL((1, DH)),
      ],
      out_specs=pl.BlockSpec(
          (B, DH // 2),
          lambda i: (jnp.where(i < NB, 0, i - NB), 0)),
      out_shape=jax.ShapeDtypeStruct((2 * NPAD, DH // 2), jnp.float32),
      scratch_shapes=[pltpu.VMEM((NPAD, DH), jnp.float32),
                      pltpu.VMEM((8, DH), jnp.float32)],
      compiler_params=_TC_PARAMS,
  )(Xp, agg, agg, Wa, ba, Wb, bb, g, be)


def _layer2(H1r, agg, Wa, ba, Wb, bb, g, be, maskp, W3, b3):
  def body(xl_ref, xr_ref, al_ref, ar_ref, wa_ref, ba_ref, wb_ref, bb_ref,
           g_ref, be_ref, m_ref, w3_ref, b3_ref, o_ref, hbuf, sums):
    i = pl.program_id(0)

    @pl.when(i < NB)
    def _():
      h0 = jnp.concatenate([xl_ref[...] + al_ref[...],
                            xr_ref[...] + ar_ref[...]], axis=1)
      hbuf[pl.ds(i * B, B), :] = _mlp_block(h0, wa_ref, ba_ref, wb_ref,
                                            bb_ref, i, sums)

    @pl.when(i >= NB)
    def _():
      r = i - NB
      scale, shift = _bn_coeffs(sums, g_ref, be_ref)
      hv = hbuf[pl.ds(r * B, B), :] * scale + shift
      hd = jnp.where(m_ref[...], hv + hv, 0.0)
      z = jnp.dot(hd.astype(jnp.bfloat16), w3_ref[...],
                  preferred_element_type=jnp.float32) + b3_ref[...]
      zmax = jnp.max(z, axis=1, keepdims=True)
      lse = jnp.log(jnp.sum(jnp.exp(z - zmax), axis=1, keepdims=True)) + zmax
      o_ref[...] = z - lse

  clamp = lambda i: (jnp.minimum(i, NB - 1), 0)
  clamp_hi = lambda i: (NB + jnp.minimum(i, NB - 1), 0)
  phase1 = lambda i: (jnp.where(i < NB, 0, i - NB), 0)
  return pl.pallas_call(
      body,
      grid=(2 * NB,),
      in_specs=[
          pl.BlockSpec((B, DH // 2), clamp),
          pl.BlockSpec((B, DH // 2), clamp_hi),
          pl.BlockSpec((B, DH // 2), clamp),
          pl.BlockSpec((B, DH // 2), clamp_hi),
          _FULL((DH, DH)), _FULL((1, DH)), _FULL((DH, DH)), _FULL((1, DH)),
          _FULL((1, DH)), _FULL((1, DH)),
          pl.BlockSpec((B, DH), phase1),
          _FULL((DH, DOUT)), _FULL((1, DOUT)),
      ],
      out_specs=pl.BlockSpec((B, DOUT), phase1),
      out_shape=jax.ShapeDtypeStruct((NPAD, DOUT), jnp.float32),
      scratch_shapes=[pltpu.VMEM((NPAD, DH), jnp.float32),
                      pltpu.VMEM((8, DH), jnp.float32)],
      compiler_params=_TC_PARAMS,
  )(H1r, H1r, agg, agg, Wa, ba, Wb, bb, g, be, maskp, W3, b3)


def kernel(X, edge_index, W1a, b1a, W1b, b1b, g1, be1,
           W2a, b2a, W2b, b2b, g2, be2, W3, b3):
  src = edge_index[0].astype(jnp.int32)
  dst = edge_index[1].astype(jnp.int32)
  pad_ids = lax.iota(jnp.int32, EPAD - E)
  srcp = jnp.concatenate([src, pad_ids % N])
  dstp = jnp.concatenate([dst, N + pad_ids % (NPAD - N)])
  src1_4 = srcp.reshape(NC, NS, CH1, CHUNK)
  dst1_4 = dstp.reshape(NC, NS, CH1, CHUNK)

  src2_4 = jnp.stack([srcp, srcp + NPAD]).reshape(NC, NS, CH2, CHUNK)
  dst2_4 = jnp.stack([dstp, dstp]).reshape(NC, NS, CH2, CHUNK)

  Xp = jnp.pad(X, ((0, NPAD - N), (0, 0)))

  b1a2, b1b2 = b1a[None, :], b1b[None, :]
  b2a2, b2b2 = b2a[None, :], b2b[None, :]
  g1r, be1r = g1[None, :], be1[None, :]
  g2r, be2r = g2[None, :], be2[None, :]
  b3r = b3[None, :]

  bf = jnp.bfloat16
  agg1 = _sc_segsum(Xp, src1_4, dst1_4, CH1)
  H1r = _layer1(Xp, agg1, W1a.astype(bf), b1a2, W1b.astype(bf), b1b2,
                g1r, be1r)

  agg2 = _sc_segsum(H1r, src2_4, dst2_4, CH2)
  mask = jax.random.bernoulli(jax.random.key(123), 0.5, (N, DH))
  maskp = jnp.pad(mask, ((0, NPAD - N), (0, 0)))
  out = _layer2(H1r, agg2, W2a.astype(bf), b2a2, W2b.astype(bf), b2b2,
                g2r, be2r, maskp, W3.astype(bf), b3r)
  return out[:N]


# speedup vs baseline: 1.2450x; 1.0819x over previous
import functools

import jax
import jax.numpy as jnp
from jax import lax
from jax.experimental import pallas as pl
from jax.experimental.pallas import tpu as pltpu
from jax.experimental.pallas import tpu_sc as plsc

N = 10000
NPAD = 10240
E = 320000
DIN = 128
DH = 256
DOUT = 64
B = 512
NB = NPAD // B
NC = 2
NS = 16
CHUNK = 128
RPS = NPAD // NS
W = 128
CH1 = 80
CH2 = 160
EPAD = NC * NS * CH1 * CHUNK
IBLK = 32


def _sc_segsum(table, src4, dst4, ch):
  mesh = plsc.VectorSubcoreMesh(core_axis_name="c", subcore_axis_name="s")

  @functools.partial(
      pl.kernel,
      out_type=jax.ShapeDtypeStruct((2 * NPAD, W), jnp.float32),
      mesh=mesh,
      scratch_types=[
          pltpu.VMEM((IBLK, CHUNK), jnp.int32),
          pltpu.VMEM((IBLK, CHUNK), jnp.int32),
          pltpu.VMEM((CHUNK, W), jnp.float32),
          pltpu.VMEM((CHUNK, W), jnp.float32),
          pltpu.VMEM_SHARED((NPAD, W), jnp.float32),
          pltpu.SemaphoreType.DMA,
          pltpu.SemaphoreType.DMA,
      ],
  )
  def k(table_hbm, src_hbm, dst_hbm, out_hbm, src_v, dst_v, rows0, rows1,
        acc, sem_g, sem_z):
    c = lax.axis_index("c")
    s = lax.axis_index("s")
    dc = c if dst_hbm.shape[0] == NC else c * 0

    def start_g(j, buf):
      pltpu.async_copy(table_hbm.at[src_v.at[j]], buf, sem_g)

    def wait_g(buf):
      pltpu.make_async_copy(table_hbm.at[src_v.at[0]], buf, sem_g).wait()

    def scat(j, buf):
      pltpu.sync_copy(buf, acc.at[dst_v.at[j]], add=True)

    def zrow(r, carry):
      for kk in range(W // 16):
        rows1[r, pl.ds(kk * 16, 16)] = jnp.zeros((16,), jnp.float32)
      return carry
    lax.fori_loop(0, CHUNK, zrow, 0)
    zdescs = [pltpu.async_copy(rows1,
                               acc.at[pl.ds(s * RPS + q * CHUNK, CHUNK)],
                               sem_z)
              for q in range(RPS // CHUNK)]
    pltpu.sync_copy(src_hbm.at[c, s, pl.ds(0, IBLK)], src_v)
    pltpu.sync_copy(dst_hbm.at[dc, s, pl.ds(0, IBLK)], dst_v)
    start_g(0, rows0)
    for zd in zdescs:
      zd.wait()
    plsc.subcore_barrier()

    for t in range(ch // IBLK):
      if t > 0:
        pltpu.sync_copy(src_hbm.at[c, s, pl.ds(t * IBLK, IBLK)], src_v)
        pltpu.sync_copy(dst_hbm.at[dc, s, pl.ds(t * IBLK, IBLK)], dst_v)
        start_g(0, rows0)

      def pair(i, carry):
        j0 = 2 * i
        wait_g(rows0)
        start_g(j0 + 1, rows1)
        scat(j0, rows0)
        wait_g(rows1)
        start_g(j0 + 2, rows0)
        scat(j0 + 1, rows1)
        return carry
      lax.fori_loop(0, IBLK // 2 - 1, pair, 0)
      wait_g(rows0)
      start_g(IBLK - 1, rows1)
      scat(IBLK - 2, rows0)
      wait_g(rows1)
      scat(IBLK - 1, rows1)
    plsc.subcore_barrier()

    pltpu.sync_copy(acc.at[pl.ds(s * RPS, RPS)],
                    out_hbm.at[pl.ds(c * NPAD + s * RPS, RPS)])

  return k(table, src4, dst4)


_TC_PARAMS = pltpu.CompilerParams(dimension_semantics=("arbitrary",))
_FULL = lambda shape: pl.BlockSpec(shape, lambda i: (0, 0))


def _mlp_block(h0, wa_ref, ba_ref, wb_ref, bb_ref, i, sums):
  h = jnp.maximum(jnp.dot(h0.astype(jnp.bfloat16), wa_ref[...],
                          preferred_element_type=jnp.float32) + ba_ref[...], 0.0)
  h = jnp.maximum(jnp.dot(h.astype(jnp.bfloat16), wb_ref[...],
                          preferred_element_type=jnp.float32) + bb_ref[...], 0.0)
  rows = i * B + lax.broadcasted_iota(jnp.int32, (B, 1), 0)
  hm = jnp.where(rows < N, h, 0.0)
  @pl.when(i == 0)
  def _():
    sums[...] = jnp.zeros_like(sums)
  sums[...] += jnp.concatenate(
      [jnp.sum(hm, 0)[None], jnp.sum(hm * hm, 0)[None],
       jnp.zeros((6, DH), jnp.float32)], axis=0)
  return h


def _bn_coeffs(sums, g_ref, be_ref):
  mean = sums[0:1, :] / N
  var = sums[1:2, :] / N - mean * mean
  scale = g_ref[...] * lax.rsqrt(var + 1e-5)
  shift = be_ref[...] - mean * scale
  return scale, shift


def _layer1(Xp, agg, Wa, ba, Wb, bb, g, be):
  def body(x_ref, aa_ref, ab_ref, wa_ref, ba_ref, wb_ref, bb_ref,
           g_ref, be_ref, o_ref, hbuf, sums):
    i = pl.program_id(0)

    @pl.when(i < NB)
    def _():
      h0 = x_ref[...] + aa_ref[...] + ab_ref[...]
      hbuf[pl.ds(i * B, B), :] = _mlp_block(h0, wa_ref, ba_ref, wb_ref,
                                            bb_ref, i, sums)

    @pl.when(i >= NB)
    def _():
      k = i - NB
      p = k // NB
      r = k % NB
      scale, shift = _bn_coeffs(sums, g_ref, be_ref)
      hv = hbuf[pl.ds(r * B, B), :] * scale + shift
      o_ref[...] = jnp.where(p == 0, hv[:, :DH // 2], hv[:, DH // 2:])

  clamp = lambda i: (jnp.minimum(i, NB - 1), 0)
  clamp_hi = lambda i: (NB + jnp.minimum(i, NB - 1), 0)
  return pl.pallas_call(
      body,
      grid=(3 * NB,),
      in_specs=[
          pl.BlockSpec((B, DIN), clamp),
          pl.BlockSpec((B, DIN), clamp),
          pl.BlockSpec((B, DIN), clamp_hi),
          _FULL((DIN, DH)), _FULL((1, DH)), _FULL((DH, DH)), _FULL((1, DH)),
          _FULL((1, DH)), _FULL((1, DH)),
      ],
      out_specs=pl.BlockSpec(
          (B, DH // 2),
          lambda i: (jnp.where(i < NB, 0, i - NB), 0)),
      out_shape=jax.ShapeDtypeStruct((2 * NPAD, DH // 2), jnp.float32),
      scratch_shapes=[pltpu.VMEM((NPAD, DH), jnp.float32),
                      pltpu.VMEM((8, DH), jnp.float32)],
      compiler_params=_TC_PARAMS,
  )(Xp, agg, agg, Wa, ba, Wb, bb, g, be)


def _layer2(H1r, agg, Wa, ba, Wb, bb, g, be, maskp, W3, b3):
  def body(xl_ref, xr_ref, al_ref, ar_ref, wa_ref, ba_ref, wb_ref, bb_ref,
           g_ref, be_ref, m_ref, w3_ref, b3_ref, o_ref, hbuf, sums):
    i = pl.program_id(0)

    @pl.when(i < NB)
    def _():
      h0 = jnp.concatenate([xl_ref[...] + al_ref[...],
                            xr_ref[...] + ar_ref[...]], axis=1)
      hbuf[pl.ds(i * B, B), :] = _mlp_block(h0, wa_ref, ba_ref, wb_ref,
                                            bb_ref, i, sums)

    @pl.when(i >= NB)
    def _():
      r = i - NB
      scale, shift = _bn_coeffs(sums, g_ref, be_ref)
      hv = hbuf[pl.ds(r * B, B), :] * scale + shift
      hd = jnp.where(m_ref[...], hv + hv, 0.0)
      z = jnp.dot(hd.astype(jnp.bfloat16), w3_ref[...],
                  preferred_element_type=jnp.float32) + b3_ref[...]
      zmax = jnp.max(z, axis=1, keepdims=True)
      lse = jnp.log(jnp.sum(jnp.exp(z - zmax), axis=1, keepdims=True)) + zmax
      o_ref[...] = z - lse

  clamp = lambda i: (jnp.minimum(i, NB - 1), 0)
  clamp_hi = lambda i: (NB + jnp.minimum(i, NB - 1), 0)
  phase1 = lambda i: (jnp.where(i < NB, 0, i - NB), 0)
  return pl.pallas_call(
      body,
      grid=(2 * NB,),
      in_specs=[
          pl.BlockSpec((B, DH // 2), clamp),
          pl.BlockSpec((B, DH // 2), clamp_hi),
          pl.BlockSpec((B, DH // 2), clamp),
          pl.BlockSpec((B, DH // 2), clamp_hi),
          _FULL((DH, DH)), _FULL((1, DH)), _FULL((DH, DH)), _FULL((1, DH)),
          _FULL((1, DH)), _FULL((1, DH)),
          pl.BlockSpec((B, DH), phase1),
          _FULL((DH, DOUT)), _FULL((1, DOUT)),
      ],
      out_specs=pl.BlockSpec((B, DOUT), phase1),
      out_shape=jax.ShapeDtypeStruct((NPAD, DOUT), jnp.float32),
      scratch_shapes=[pltpu.VMEM((NPAD, DH), jnp.float32),
                      pltpu.VMEM((8, DH), jnp.float32)],
      compiler_params=_TC_PARAMS,
  )(H1r, H1r, agg, agg, Wa, ba, Wb, bb, g, be, maskp, W3, b3)


def kernel(X, edge_index, W1a, b1a, W1b, b1b, g1, be1,
           W2a, b2a, W2b, b2b, g2, be2, W3, b3):
  src = edge_index[0].astype(jnp.int32)
  dst = edge_index[1].astype(jnp.int32)
  pad_ids = lax.iota(jnp.int32, EPAD - E)
  srcp = jnp.concatenate([src, pad_ids % N])
  dstp = jnp.concatenate([dst, N + pad_ids % (NPAD - N)])
  src1_4 = srcp.reshape(NC, NS, CH1, CHUNK)
  dst1_4 = dstp.reshape(NC, NS, CH1, CHUNK)

  src2_4 = jnp.stack([srcp, srcp + NPAD]).reshape(NC, NS, CH2, CHUNK)
  dst2_4 = dstp.reshape(1, NS, CH2, CHUNK)

  Xp = jnp.pad(X, ((0, NPAD - N), (0, 0)))

  b1a2, b1b2 = b1a[None, :], b1b[None, :]
  b2a2, b2b2 = b2a[None, :], b2b[None, :]
  g1r, be1r = g1[None, :], be1[None, :]
  g2r, be2r = g2[None, :], be2[None, :]
  b3r = b3[None, :]

  bf = jnp.bfloat16
  agg1 = _sc_segsum(Xp, src1_4, dst1_4, CH1)
  H1r = _layer1(Xp, agg1, W1a.astype(bf), b1a2, W1b.astype(bf), b1b2,
                g1r, be1r)

  agg2 = _sc_segsum(H1r, src2_4, dst2_4, CH2)
  mask = jax.random.bernoulli(jax.random.key(123), 0.5, (N, DH))
  maskp = jnp.pad(mask, ((0, NPAD - N), (0, 0)))
  out = _layer2(H1r, agg2, W2a.astype(bf), b2a2, W2b.astype(bf), b2b2,
                g2r, be2r, maskp, W3.astype(bf), b3r)
  return out[:N]
